# baseline (speedup 1.0000x reference)
import jax
import jax.numpy as jnp
from jax.experimental import pallas as pl
from jax.experimental.pallas import tpu as pltpu

D = 2048
BJ = 128
NJ = D // BJ
NBUF = 12


def _fused_kernel(x_ref, wgr_ref, bgr_ref, wgi_ref, bgi_ref,
                  wr_hbm, wi_hbm, br_ref, bi_ref,
                  out_ref, mir_ref, mii_ref,
                  wr_buf, wi_buf, rsem, isem):
    x = x_ref[...]

    def route(wg_ref, bg_ref):
        s = [jnp.maximum(jnp.sum(x * wg_ref[e:e + 1, :]) + bg_ref[e], 0.0)
             for e in range(4)]
        i1 = jnp.where(s[0] >= s[1], 0, 1)
        s1 = jnp.maximum(s[0], s[1])
        i2 = jnp.where(s[2] >= s[3], 2, 3)
        s2 = jnp.maximum(s[2], s[3])
        m = jnp.maximum(s1, s2)
        e1 = jnp.exp(jnp.broadcast_to(s1 - m, (1, 1)))
        e2 = jnp.exp(jnp.broadcast_to(s2 - m, (1, 1)))
        denom = e1 + e2
        p1 = e1 / denom
        p2 = e2 / denom
        mi = jnp.where(p1 >= p2, 0, 1).astype(jnp.int32)
        return i1, i2, p1, p2, mi

    ir1, ir2, pr1, pr2, mir = route(wgr_ref, bgr_ref)
    ii1, ii2, pi1, pi2, mii = route(wgi_ref, bgi_ref)
    mir_ref[...] = mir
    mii_ref[...] = mii

    lanes = jax.lax.broadcasted_iota(jnp.int32, (4, 1), 0)
    w_r = (jnp.where(lanes == ir1, pr1, 0.0) +
           jnp.where(lanes == ir2, pr2, 0.0))
    w_i = (jnp.where(lanes == ii1, pi1, 0.0) +
           jnp.where(lanes == ii2, pi2, 0.0))
    out_ref[...] = (jnp.sum(w_r * br_ref[...], axis=0, keepdims=True) +
                    jnp.sum(w_i * bi_ref[...], axis=0, keepdims=True))

    steps = [(j, k) for j in range(NJ) for k in range(2)]
    e_r = [ir1, ir2]
    e_i = [ii1, ii2]
    p_r = [pr1, pr2]
    p_i = [pi1, pi2]

    def copies(t, b):
        j, k = steps[t]
        src_r = wr_hbm.at[e_r[k], pl.ds(j * BJ, BJ), :]
        src_i = wi_hbm.at[e_i[k], pl.ds(j * BJ, BJ), :]
        return (pltpu.make_async_copy(src_r, wr_buf.at[b], rsem.at[b]),
                pltpu.make_async_copy(src_i, wi_buf.at[b], isem.at[b]))

    for t in range(min(NBUF, len(steps))):
        cr, ci = copies(t, t % NBUF)
        cr.start()
        ci.start()

    dn = (((1,), (1,)), ((), ()))
    for t, (j, k) in enumerate(steps):
        b = t % NBUF
        cr, ci = copies(t, b)
        cr.wait()
        ci.wait()
        yr = jax.lax.dot_general(x, wr_buf[b], dn,
                                 precision=jax.lax.Precision.HIGHEST,
                                 preferred_element_type=jnp.float32)
        yi = jax.lax.dot_general(x, wi_buf[b], dn,
                                 precision=jax.lax.Precision.HIGHEST,
                                 preferred_element_type=jnp.float32)
        out_ref[:, pl.ds(j * BJ, BJ)] += p_r[k] * yr + p_i[k] * yi
        nxt = t + NBUF
        if nxt < len(steps):
            nr, ni = copies(nxt, nxt % NBUF)
            nr.start()
            ni.start()


@jax.jit
def kernel(rgb_local, ir_local, W_gate_rgb, b_gate_rgb, W_gate_ir, b_gate_ir,
           W_exp_rgb, b_exp_rgb, W_exp_ir, b_exp_ir):
    B = rgb_local.shape[0]
    x = jnp.concatenate(
        [rgb_local.reshape(B, -1), ir_local.reshape(B, -1)], axis=1)

    combined, max_idx_rgb, max_idx_ir = pl.pallas_call(
        _fused_kernel,
        in_specs=[
            pl.BlockSpec(memory_space=pltpu.VMEM),
            pl.BlockSpec(memory_space=pltpu.VMEM),
            pl.BlockSpec(memory_space=pltpu.SMEM),
            pl.BlockSpec(memory_space=pltpu.VMEM),
            pl.BlockSpec(memory_space=pltpu.SMEM),
            pl.BlockSpec(memory_space=pl.ANY),
            pl.BlockSpec(memory_space=pl.ANY),
            pl.BlockSpec(memory_space=pltpu.VMEM),
            pl.BlockSpec(memory_space=pltpu.VMEM),
        ],
        out_specs=(
            pl.BlockSpec(memory_space=pltpu.VMEM),
            pl.BlockSpec(memory_space=pltpu.VMEM),
            pl.BlockSpec(memory_space=pltpu.VMEM),
        ),
        out_shape=(
            jax.ShapeDtypeStruct((1, D), jnp.float32),
            jax.ShapeDtypeStruct((1, 1), jnp.int32),
            jax.ShapeDtypeStruct((1, 1), jnp.int32),
        ),
        scratch_shapes=[
            pltpu.VMEM((NBUF, BJ, D), jnp.float32),
            pltpu.VMEM((NBUF, BJ, D), jnp.float32),
            pltpu.SemaphoreType.DMA((NBUF,)),
            pltpu.SemaphoreType.DMA((NBUF,)),
        ],
    )(x, W_gate_rgb, b_gate_rgb, W_gate_ir, b_gate_ir,
      W_exp_rgb, W_exp_ir, b_exp_rgb, b_exp_ir)

    return (combined, max_idx_rgb.reshape(1), max_idx_ir.reshape(1))


# speedup vs baseline: 2.5926x; 2.5926x over previous
import jax
import jax.numpy as jnp
from jax.experimental import pallas as pl
from jax.experimental.pallas import tpu as pltpu

D = 2048
BJ = 128
NJ = D // BJ
NBUF = 12


def _fused_kernel(x_ref, wgr_ref, bgr_ref, wgi_ref, bgi_ref,
                  wr_hbm, wi_hbm, br_ref, bi_ref,
                  out_ref, mir_ref, mii_ref,
                  wr_buf, wi_buf, rsem, isem):
    x = x_ref[...]

    def route(wg_ref, bg_ref):
        s = [jnp.maximum(jnp.sum(x * wg_ref[e:e + 1, :]) + bg_ref[e], 0.0)
             for e in range(4)]
        i1 = jnp.where(s[0] >= s[1], 0, 1)
        s1 = jnp.maximum(s[0], s[1])
        i2 = jnp.where(s[2] >= s[3], 2, 3)
        s2 = jnp.maximum(s[2], s[3])
        m = jnp.maximum(s1, s2)
        e1 = jnp.exp(jnp.broadcast_to(s1 - m, (1, 1)))
        e2 = jnp.exp(jnp.broadcast_to(s2 - m, (1, 1)))
        denom = e1 + e2
        p1 = e1 / denom
        p2 = e2 / denom
        mi = jnp.where(p1 >= p2, 0, 1).astype(jnp.int32)
        return i1, i2, p1, p2, mi

    ir1, ir2, pr1, pr2, mir = route(wgr_ref, bgr_ref)
    ii1, ii2, pi1, pi2, mii = route(wgi_ref, bgi_ref)
    mir_ref[...] = mir
    mii_ref[...] = mii

    lanes = jax.lax.broadcasted_iota(jnp.int32, (4, 1), 0)
    w_r = (jnp.where(lanes == ir1, pr1, 0.0) +
           jnp.where(lanes == ir2, pr2, 0.0))
    w_i = (jnp.where(lanes == ii1, pi1, 0.0) +
           jnp.where(lanes == ii2, pi2, 0.0))
    out_ref[...] = (jnp.sum(w_r * br_ref[...], axis=0, keepdims=True) +
                    jnp.sum(w_i * bi_ref[...], axis=0, keepdims=True))

    steps = [(j, k) for j in range(NJ) for k in range(2)]
    e_r = [ir1, ir2]
    e_i = [ii1, ii2]
    p_r = [pr1, pr2]
    p_i = [pi1, pi2]

    def copies(t, b):
        j, k = steps[t]
        src_r = wr_hbm.at[e_r[k], pl.ds(j * BJ, BJ), :]
        src_i = wi_hbm.at[e_i[k], pl.ds(j * BJ, BJ), :]
        return (pltpu.make_async_copy(src_r, wr_buf.at[b], rsem.at[b]),
                pltpu.make_async_copy(src_i, wi_buf.at[b], isem.at[b]))

    for t in range(min(NBUF, len(steps))):
        cr, ci = copies(t, t % NBUF)
        cr.start()
        ci.start()

    dn = (((1,), (1,)), ((), ()))
    for t, (j, k) in enumerate(steps):
        b = t % NBUF
        cr, ci = copies(t, b)
        cr.wait()
        ci.wait()
        yr = jax.lax.dot_general(x, wr_buf[b], dn,
                                 preferred_element_type=jnp.float32)
        yi = jax.lax.dot_general(x, wi_buf[b], dn,
                                 preferred_element_type=jnp.float32)
        out_ref[:, pl.ds(j * BJ, BJ)] += p_r[k] * yr + p_i[k] * yi
        nxt = t + NBUF
        if nxt < len(steps):
            nr, ni = copies(nxt, nxt % NBUF)
            nr.start()
            ni.start()


@jax.jit
def kernel(rgb_local, ir_local, W_gate_rgb, b_gate_rgb, W_gate_ir, b_gate_ir,
           W_exp_rgb, b_exp_rgb, W_exp_ir, b_exp_ir):
    B = rgb_local.shape[0]
    x = jnp.concatenate(
        [rgb_local.reshape(B, -1), ir_local.reshape(B, -1)], axis=1)

    combined, max_idx_rgb, max_idx_ir = pl.pallas_call(
        _fused_kernel,
        in_specs=[
            pl.BlockSpec(memory_space=pltpu.VMEM),
            pl.BlockSpec(memory_space=pltpu.VMEM),
            pl.BlockSpec(memory_space=pltpu.SMEM),
            pl.BlockSpec(memory_space=pltpu.VMEM),
            pl.BlockSpec(memory_space=pltpu.SMEM),
            pl.BlockSpec(memory_space=pl.ANY),
            pl.BlockSpec(memory_space=pl.ANY),
            pl.BlockSpec(memory_space=pltpu.VMEM),
            pl.BlockSpec(memory_space=pltpu.VMEM),
        ],
        out_specs=(
            pl.BlockSpec(memory_space=pltpu.VMEM),
            pl.BlockSpec(memory_space=pltpu.VMEM),
            pl.BlockSpec(memory_space=pltpu.VMEM),
        ),
        out_shape=(
            jax.ShapeDtypeStruct((1, D), jnp.float32),
            jax.ShapeDtypeStruct((1, 1), jnp.int32),
            jax.ShapeDtypeStruct((1, 1), jnp.int32),
        ),
        scratch_shapes=[
            pltpu.VMEM((NBUF, BJ, D), jnp.float32),
            pltpu.VMEM((NBUF, BJ, D), jnp.float32),
            pltpu.SemaphoreType.DMA((NBUF,)),
            pltpu.SemaphoreType.DMA((NBUF,)),
        ],
    )(x, W_gate_rgb, b_gate_rgb, W_gate_ir, b_gate_ir,
      W_exp_rgb, W_exp_ir, b_exp_rgb, b_exp_ir)

    return (combined, max_idx_rgb.reshape(1), max_idx_ir.reshape(1))
